# baseline (speedup 1.0000x reference)
import functools

import jax
import jax.numpy as jnp
from jax import lax
from jax.experimental import pallas as pl
from jax.experimental.pallas import tpu as pltpu
from jax.experimental.pallas import tpu_sc as plsc

E = 8
TOP_K = 2
D = 1024
DFF = 4096
T = 2048
A = T * TOP_K
BT = 256
G = A // BT + E
P = G * BT
NW = 32

_INV_SQRT2 = 0.7071067811865476


def _gelu(v):
    return 0.5 * v * (1.0 + jax.lax.erf(v * _INV_SQRT2))


def _router_body(x_ref, gw_ref, gb_ref, logits_ref, sel_ref, w_ref):
    h = x_ref[...]
    logits = jnp.dot(h, gw_ref[...], preferred_element_type=jnp.float32)
    logits = logits + gb_ref[...]
    logits_ref[...] = logits
    e_iota = jax.lax.broadcasted_iota(jnp.int32, (T, E), 1)
    m0 = jnp.max(logits, axis=1, keepdims=True)
    s0 = jnp.min(jnp.where(logits == m0, e_iota, E), axis=1, keepdims=True)
    masked = jnp.where(e_iota == s0, -jnp.inf, logits)
    m1 = jnp.max(masked, axis=1, keepdims=True)
    s1 = jnp.min(jnp.where(masked == m1, e_iota, E), axis=1, keepdims=True)
    t = jnp.exp(m1 - m0)
    w0 = 1.0 / (1.0 + t)
    w1 = 1.0 - w0
    pair = jax.lax.broadcasted_iota(jnp.int32, (T, 2), 1)
    sel_ref[...] = jnp.where(pair == 0, s0, s1)
    w_ref[...] = jnp.where(pair == 0, w0, w1)


def _router(x2d, gate_w, gate_b):
    return pl.pallas_call(
        _router_body,
        out_shape=(
            jax.ShapeDtypeStruct((T, E), jnp.float32),
            jax.ShapeDtypeStruct((T, 2), jnp.int32),
            jax.ShapeDtypeStruct((T, 2), jnp.float32),
        ),
    )(x2d, gate_w, gate_b.reshape(1, E))


def _dispatch(sel, wts):
    a_e = sel.reshape(-1)
    a_w = wts.reshape(-1)
    onehot = (a_e[:, None] == jnp.arange(E, dtype=jnp.int32)[None, :])
    onehot = onehot.astype(jnp.int32)
    cum = jnp.cumsum(onehot, axis=0)
    n_e = cum[-1]
    rank = jnp.sum(onehot * (cum - onehot), axis=1)
    nblk = (n_e + BT - 1) // BT
    cblk = jnp.cumsum(nblk)
    off_blk = jnp.concatenate([jnp.zeros(1, cblk.dtype), cblk[:-1]])
    pos = (off_blk[a_e] * BT + rank).astype(jnp.int32)
    tok = (jnp.arange(A, dtype=jnp.int32) // TOP_K)
    pad_rows = (jnp.arange(P, dtype=jnp.int32) % T).astype(jnp.float32)
    base_init = jnp.stack([pad_rows, jnp.zeros((P,), jnp.float32)], axis=1)
    packed = base_init.at[pos].set(
        jnp.stack([tok.astype(jnp.float32), a_w], axis=1))
    idx_gather = packed[:, 0].astype(jnp.int32)
    w_sorted = packed[:, 1:2]
    total = cblk[-1]
    gids = jnp.arange(G, dtype=cblk.dtype)
    eb = jnp.searchsorted(cblk, gids, side="right")
    valid = (gids < total).astype(jnp.int32)
    last = jnp.maximum(total - 1, 0)
    we_idx = jnp.where(valid == 1, jnp.minimum(eb, E - 1), eb[last])
    we_idx = we_idx.astype(jnp.int32)
    xb_idx = jnp.where(valid == 1, gids, last).astype(jnp.int32)
    pos_ab = jnp.concatenate([pos[0::2], pos[1::2]])
    return idx_gather, w_sorted, we_idx, xb_idx, valid, pos_ab


@functools.lru_cache(maxsize=None)
def _make_row_gather(n_out, chunk):
    per_w = n_out // NW
    assert per_w % chunk == 0 and per_w % 8 == 0
    nch = per_w // chunk
    mesh = plsc.VectorSubcoreMesh(core_axis_name="c", subcore_axis_name="s")

    @functools.partial(
        pl.kernel,
        mesh=mesh,
        out_type=jax.ShapeDtypeStruct((n_out, D), jnp.float32),
        scratch_types=[
            pltpu.VMEM((per_w,), jnp.int32),
            pltpu.VMEM((chunk, D), jnp.float32),
            pltpu.VMEM((chunk, D), jnp.float32),
            pltpu.SemaphoreType.DMA,
            pltpu.SemaphoreType.DMA,
            pltpu.SemaphoreType.DMA,
            pltpu.SemaphoreType.DMA,
        ],
    )
    def gather_k(src_hbm, idx_hbm, out_hbm, idx_v, buf0, buf1, g0, g1, s0, s1):
        wid = lax.axis_index("s") * 2 + lax.axis_index("c")
        base = wid * per_w
        bufs = (buf0, buf1)
        gsem = (g0, g1)
        ssem = (s0, s1)
        pltpu.sync_copy(idx_hbm.at[pl.ds(base, per_w)], idx_v)

        def gstart(c):
            b = c & 1
            return pltpu.async_copy(
                src_hbm.at[idx_v.at[pl.ds(c * chunk, chunk)]], bufs[b], gsem[b])

        gathers = {0: gstart(0)}
        stores = {}
        for c in range(nch):
            b = c & 1
            if c + 1 < nch:
                if c - 1 >= 0:
                    stores[c - 1].wait()
                gathers[c + 1] = gstart(c + 1)
            gathers[c].wait()
            stores[c] = pltpu.async_copy(
                bufs[b], out_hbm.at[pl.ds(base + c * chunk, chunk)], ssem[b])
        stores[nch - 1].wait()
        if nch >= 2:
            stores[nch - 2].wait()

    return gather_k


DH = D // 2


def _moe_body(we_ref, xb_ref, vld_ref, x_ref, w1_ref, b1_ref, w2_ref, b2_ref,
              ws_ref, out_ref, h1_ref):
    g = pl.program_id(0)
    d = pl.program_id(1)

    @pl.when((vld_ref[g] == 1) & (d == 0))
    def _():
        h1 = jnp.dot(x_ref[...], w1_ref[0], preferred_element_type=jnp.float32)
        h1_ref[...] = _gelu(h1 + b1_ref[0])

    @pl.when(vld_ref[g] == 1)
    def _():
        y = jnp.dot(h1_ref[...], w2_ref[0], preferred_element_type=jnp.float32)
        out_ref[...] = (y + b2_ref[0]) * ws_ref[...]


def _moe_ffn(x_sorted, w_sorted, we_idx, xb_idx, valid, w1, b1, w2, b2):
    grid_spec = pltpu.PrefetchScalarGridSpec(
        num_scalar_prefetch=3,
        grid=(G, 2),
        in_specs=[
            pl.BlockSpec((BT, D), lambda g, d, we, xb, vld: (xb[g], 0)),
            pl.BlockSpec((1, D, DFF), lambda g, d, we, xb, vld: (we[g], 0, 0)),
            pl.BlockSpec((1, 1, DFF), lambda g, d, we, xb, vld: (we[g], 0, 0)),
            pl.BlockSpec((1, DFF, DH), lambda g, d, we, xb, vld: (we[g], 0, d)),
            pl.BlockSpec((1, 1, DH), lambda g, d, we, xb, vld: (we[g], 0, d)),
            pl.BlockSpec((BT, 1), lambda g, d, we, xb, vld: (xb[g], 0)),
        ],
        out_specs=pl.BlockSpec((BT, DH), lambda g, d, we, xb, vld: (g, d)),
        scratch_shapes=[pltpu.VMEM((BT, DFF), jnp.float32)],
    )
    return pl.pallas_call(
        _moe_body,
        grid_spec=grid_spec,
        out_shape=jax.ShapeDtypeStruct((P, D), jnp.float32),
        compiler_params=pltpu.CompilerParams(
            dimension_semantics=("arbitrary", "arbitrary"),
            vmem_limit_bytes=120 * 1024 * 1024,
        ),
    )(we_idx, xb_idx, valid, x_sorted, w1, b1.reshape(E, 1, DFF), w2,
      b2.reshape(E, 1, D), w_sorted)


def _add_body(a_ref, b_ref, o_ref):
    o_ref[...] = a_ref[...] + b_ref[...]


def _pair_add(ab):
    nblk = 4
    return pl.pallas_call(
        _add_body,
        grid=(nblk,),
        in_specs=[
            pl.BlockSpec((T // nblk, D), lambda i: (i, 0)),
            pl.BlockSpec((T // nblk, D), lambda i: (i + nblk, 0)),
        ],
        out_specs=pl.BlockSpec((T // nblk, D), lambda i: (i, 0)),
        out_shape=jax.ShapeDtypeStruct((T, D), jnp.float32),
    )(ab, ab)


@jax.jit
def kernel(x, gate_w, gate_b, w1, b1, w2, b2):
    bsz, seq, dim = x.shape
    h = x.reshape(-1, dim)
    logits, sel, wts = _router(h, gate_w, gate_b)
    idx_gather, w_sorted, we_idx, xb_idx, valid, pos_ab = _dispatch(sel, wts)
    x_sorted = _make_row_gather(P, 48)(h, idx_gather)
    y = _moe_ffn(x_sorted, w_sorted, we_idx, xb_idx, valid, w1, b1, w2, b2)
    out_ab = _make_row_gather(2 * T, 32)(y, pos_ab)
    final = _pair_add(out_ab)
    return final.reshape(bsz, seq, dim), logits


# speedup vs baseline: 1.2929x; 1.2929x over previous
import functools

import jax
import jax.numpy as jnp
from jax import lax
from jax.experimental import pallas as pl
from jax.experimental.pallas import tpu as pltpu
from jax.experimental.pallas import tpu_sc as plsc

E = 8
TOP_K = 2
D = 1024
DFF = 4096
T = 2048
A = T * TOP_K
BT = 256
G = A // BT + E
P = G * BT
NW = 32

_INV_SQRT2 = 0.7071067811865476


def _gelu(v):
    return 0.5 * v * (1.0 + jax.lax.erf(v * _INV_SQRT2))


def _router_body(x_ref, gw_ref, gb_ref, logits_ref, sel_ref, w_ref, rank_ref,
                 cnt_ref):
    h = x_ref[...]
    logits = jnp.dot(h, gw_ref[...], preferred_element_type=jnp.float32)
    logits = logits + gb_ref[...]
    logits_ref[...] = logits
    e_iota = jax.lax.broadcasted_iota(jnp.int32, (T, E), 1)
    m0 = jnp.max(logits, axis=1, keepdims=True)
    s0 = jnp.min(jnp.where(logits == m0, e_iota, E), axis=1, keepdims=True)
    masked = jnp.where(e_iota == s0, -jnp.inf, logits)
    m1 = jnp.max(masked, axis=1, keepdims=True)
    s1 = jnp.min(jnp.where(masked == m1, e_iota, E), axis=1, keepdims=True)
    t = jnp.exp(m1 - m0)
    w0 = 1.0 / (1.0 + t)
    w1 = 1.0 - w0
    pair = jax.lax.broadcasted_iota(jnp.int32, (T, 2), 1)
    sel_ref[...] = jnp.where(pair == 0, s0, s1)
    w_ref[...] = jnp.where(pair == 0, w0, w1)
    cnt = (jnp.where(e_iota == s0, 1.0, 0.0)
           + jnp.where(e_iota == s1, 1.0, 0.0))
    r_iota = jax.lax.broadcasted_iota(jnp.int32, (T, T), 0)
    c_iota = jax.lax.broadcasted_iota(jnp.int32, (T, T), 1)
    tri = jnp.where(c_iota < r_iota, 1.0, 0.0)
    c_lt = jnp.dot(tri, cnt, preferred_element_type=jnp.float32)
    r0 = jnp.sum(jnp.where(e_iota == s0, c_lt, 0.0), axis=1, keepdims=True)
    r1 = jnp.sum(jnp.where(e_iota == s1, c_lt, 0.0), axis=1, keepdims=True)
    rank_ref[...] = jnp.where(pair == 0, r0, r1).astype(jnp.int32)
    cnt_ref[...] = jnp.sum(cnt, axis=0, keepdims=True).astype(jnp.int32)


def _router(x2d, gate_w, gate_b):
    return pl.pallas_call(
        _router_body,
        out_shape=(
            jax.ShapeDtypeStruct((T, E), jnp.float32),
            jax.ShapeDtypeStruct((T, 2), jnp.int32),
            jax.ShapeDtypeStruct((T, 2), jnp.float32),
            jax.ShapeDtypeStruct((T, 2), jnp.int32),
            jax.ShapeDtypeStruct((1, E), jnp.int32),
        ),
        compiler_params=pltpu.CompilerParams(
            vmem_limit_bytes=120 * 1024 * 1024),
    )(x2d, gate_w, gate_b.reshape(1, E))


def _dispatch(sel, wts, rank, n_e):
    a_e = sel.reshape(-1)
    a_w = wts.reshape(-1)
    rank = rank.reshape(-1)
    nblk = (n_e + BT - 1) // BT
    cblk = jnp.cumsum(nblk)
    off_blk = jnp.concatenate([jnp.zeros(1, cblk.dtype), cblk[:-1]])
    onehot = (a_e[:, None] == jnp.arange(E, dtype=jnp.int32)[None, :])
    off_a = jnp.sum(onehot * off_blk[None, :], axis=1)
    pos = (off_a * BT + rank).astype(jnp.int32)
    tok = (jnp.arange(A, dtype=jnp.int32) // TOP_K)
    pad_rows = (jnp.arange(P, dtype=jnp.int32) % T).astype(jnp.float32)
    base_init = jnp.stack([pad_rows, jnp.zeros((P,), jnp.float32)], axis=1)
    packed = base_init.at[pos].set(
        jnp.stack([tok.astype(jnp.float32), a_w], axis=1))
    idx_gather = packed[:, 0].astype(jnp.int32)
    w_sorted = packed[:, 1:2]
    total = cblk[-1]
    gids = jnp.arange(G, dtype=cblk.dtype)
    eb = jnp.searchsorted(cblk, gids, side="right")
    valid = (gids < total).astype(jnp.int32)
    last = jnp.maximum(total - 1, 0)
    we_idx = jnp.where(valid == 1, jnp.minimum(eb, E - 1), eb[last])
    we_idx = we_idx.astype(jnp.int32)
    xb_idx = jnp.where(valid == 1, gids, last).astype(jnp.int32)
    pos_ab = jnp.concatenate([pos[0::2], pos[1::2]])
    return idx_gather, w_sorted, we_idx, xb_idx, valid, pos_ab


@functools.lru_cache(maxsize=None)
def _make_row_gather(n_out, chunk):
    per_w = n_out // NW
    assert per_w % chunk == 0 and per_w % 8 == 0
    nch = per_w // chunk
    mesh = plsc.VectorSubcoreMesh(core_axis_name="c", subcore_axis_name="s")

    @functools.partial(
        pl.kernel,
        mesh=mesh,
        out_type=jax.ShapeDtypeStruct((n_out, D), jnp.float32),
        scratch_types=[
            pltpu.VMEM((per_w,), jnp.int32),
            pltpu.VMEM((chunk, D), jnp.float32),
            pltpu.VMEM((chunk, D), jnp.float32),
            pltpu.SemaphoreType.DMA,
            pltpu.SemaphoreType.DMA,
            pltpu.SemaphoreType.DMA,
            pltpu.SemaphoreType.DMA,
        ],
    )
    def gather_k(src_hbm, idx_hbm, out_hbm, idx_v, buf0, buf1, g0, g1, s0, s1):
        wid = lax.axis_index("s") * 2 + lax.axis_index("c")
        base = wid * per_w
        bufs = (buf0, buf1)
        gsem = (g0, g1)
        ssem = (s0, s1)
        pltpu.sync_copy(idx_hbm.at[pl.ds(base, per_w)], idx_v)

        def gstart(c):
            b = c & 1
            return pltpu.async_copy(
                src_hbm.at[idx_v.at[pl.ds(c * chunk, chunk)]], bufs[b], gsem[b])

        gathers = {0: gstart(0)}
        stores = {}
        for c in range(nch):
            b = c & 1
            if c + 1 < nch:
                if c - 1 >= 0:
                    stores[c - 1].wait()
                gathers[c + 1] = gstart(c + 1)
            gathers[c].wait()
            stores[c] = pltpu.async_copy(
                bufs[b], out_hbm.at[pl.ds(base + c * chunk, chunk)], ssem[b])
        stores[nch - 1].wait()
        if nch >= 2:
            stores[nch - 2].wait()

    return gather_k


def _moe_body(we_ref, xb_ref, vld_ref, x_ref, w1_ref, b1_ref, w2_ref, b2_ref,
              ws_ref, out_ref):
    g = pl.program_id(0)

    @pl.when(vld_ref[g] == 1)
    def _():
        h1 = jnp.dot(x_ref[...], w1_ref[0], preferred_element_type=jnp.float32)
        h1 = _gelu(h1 + b1_ref[0])
        y = jnp.dot(h1, w2_ref[0], preferred_element_type=jnp.float32)
        out_ref[...] = (y + b2_ref[0]) * ws_ref[...]


def _moe_ffn(x_sorted, w_sorted, we_idx, xb_idx, valid, w1, b1, w2, b2):
    grid_spec = pltpu.PrefetchScalarGridSpec(
        num_scalar_prefetch=3,
        grid=(G,),
        in_specs=[
            pl.BlockSpec((BT, D), lambda g, we, xb, vld: (xb[g], 0)),
            pl.BlockSpec((1, D, DFF), lambda g, we, xb, vld: (we[g], 0, 0)),
            pl.BlockSpec((1, 1, DFF), lambda g, we, xb, vld: (we[g], 0, 0),
                         pipeline_mode=pl.Buffered(buffer_count=1)),
            pl.BlockSpec((1, DFF, D), lambda g, we, xb, vld: (we[g], 0, 0),
                         pipeline_mode=pl.Buffered(buffer_count=1)),
            pl.BlockSpec((1, 1, D), lambda g, we, xb, vld: (we[g], 0, 0),
                         pipeline_mode=pl.Buffered(buffer_count=1)),
            pl.BlockSpec((BT, 1), lambda g, we, xb, vld: (xb[g], 0)),
        ],
        out_specs=pl.BlockSpec((BT, D), lambda g, we, xb, vld: (g, 0)),
    )
    return pl.pallas_call(
        _moe_body,
        grid_spec=grid_spec,
        out_shape=jax.ShapeDtypeStruct((P, D), jnp.float32),
        compiler_params=pltpu.CompilerParams(
            dimension_semantics=("arbitrary",),
            vmem_limit_bytes=120 * 1024 * 1024,
        ),
    )(we_idx, xb_idx, valid, x_sorted, w1, b1.reshape(E, 1, DFF), w2,
      b2.reshape(E, 1, D), w_sorted)


def _add_body(a_ref, b_ref, o_ref):
    o_ref[...] = a_ref[...] + b_ref[...]


def _pair_add(ab):
    nblk = 4
    return pl.pallas_call(
        _add_body,
        grid=(nblk,),
        in_specs=[
            pl.BlockSpec((T // nblk, D), lambda i: (i, 0)),
            pl.BlockSpec((T // nblk, D), lambda i: (i + nblk, 0)),
        ],
        out_specs=pl.BlockSpec((T // nblk, D), lambda i: (i, 0)),
        out_shape=jax.ShapeDtypeStruct((T, D), jnp.float32),
    )(ab, ab)


@jax.jit
def kernel(x, gate_w, gate_b, w1, b1, w2, b2):
    bsz, seq, dim = x.shape
    h = x.reshape(-1, dim)
    logits, sel, wts, rank, cnt = _router(h, gate_w, gate_b)
    idx_gather, w_sorted, we_idx, xb_idx, valid, pos_ab = _dispatch(
        sel, wts, rank, cnt[0])
    x_sorted = _make_row_gather(P, 48)(h, idx_gather)
    y = _moe_ffn(x_sorted, w_sorted, we_idx, xb_idx, valid, w1, b1, w2, b2)
    out_ab = _make_row_gather(2 * T, 32)(y, pos_ab)
    final = _pair_add(out_ab)
    return final.reshape(bsz, seq, dim), logits
